# baseline (speedup 1.0000x reference)
import functools

import numpy as np

import jax
import jax.numpy as jnp
from jax import lax
from jax.experimental import pallas as pl
from jax.experimental.pallas import tpu as pltpu
from jax.experimental.pallas import tpu_sc as plsc

_B, _L, _C = 2, 2048, 1024
_H, _K = 16, 64
_S = 33
_HALF = 16
_MAXF, _MINF = 16.0, 1.0
_MAXR = _HALF * _MAXF
_BL = _B * _L
_T = 256
_NBLK = _BL // _T

_NW = 32
_RPW = _BL // _NW


def _prep_body(blk0, x_ref, wwt_ref, bw_ref, wkt_ref, bk_ref, kern_ref, idx_ref):
    i = blk0 + pl.program_id(0)
    b = i // (_L // _T)
    l0 = (i % (_L // _T)) * _T
    xb = x_ref[...]
    wave = jnp.dot(xb, wwt_ref[...], preferred_element_type=jnp.float32) + bw_ref[...]
    wave = wave * jax.nn.sigmoid(wave)
    freq = jax.nn.sigmoid(wave[:, :_H]) * (_MAXF - _MINF) + _MINF
    phase = jnp.tanh(wave[:, _H:]) * _MAXF

    def _hmean(v):
        w2 = _H // 2
        while w2 >= 1:
            v = v[:, :w2] + v[:, w2:]
            w2 //= 2
        return v * (1.0 / _H)

    fa = _hmean(freq)
    pa = _hmean(phase)
    km = jnp.dot(xb, wkt_ref[...], preferred_element_type=jnp.float32) + bk_ref[...]
    km = km * jax.nn.sigmoid(km)
    centers = (l0 + lax.broadcasted_iota(jnp.int32, (_T, 1), 0)).astype(jnp.float32)
    svec = lax.broadcasted_iota(jnp.int32, (_T, _S), 1).astype(jnp.float32) - float(_HALF)
    rel = svec * fa + pa
    pos = (centers + svec * fa) + pa
    validf = ((pos >= 0.0) & (pos < float(_L))).astype(jnp.float32)
    idx_ref[...] = jnp.clip(pos.astype(jnp.int32), 0, _L - 1) + b * _L
    fidx = jnp.clip((rel + _MAXR) / (2.0 * _MAXR), 0.0, 1.0) * float(_K - 1)
    ifl = jnp.clip(fidx.astype(jnp.int32), 0, _K - 2)
    wc = fidx - ifl.astype(jnp.float32)
    wf = 1.0 - wc
    repidx = lax.broadcasted_iota(jnp.int32, (_T, _S * 16), 1) // 16
    for h in range(_H):
        km_h = km[:, h * _K:(h + 1) * _K]
        kf = jnp.take_along_axis(km_h, ifl, axis=1)
        kc = jnp.take_along_axis(km_h, ifl + 1, axis=1)
        kh = (kf * wf + kc * wc) * validf
        r = jnp.concatenate([kh, jnp.zeros((_T, 64 - _S), jnp.float32)], axis=1)
        w2 = 32
        while w2 >= 1:
            r = r[:, :w2] + r[:, w2:]
            w2 //= 2
        den = r + 1e-8
        khn = kh / den
        kern_ref[:, h * _S * 16:(h + 1) * _S * 16] = jnp.take_along_axis(
            khn, repidx, axis=1)


def _prep_call(x2d, wwt, bw2, wkt, bk2, blk0, nblk):
    nrows = nblk * _T
    return pl.pallas_call(
        functools.partial(_prep_body, blk0),
        grid=(nblk,),
        in_specs=[
            pl.BlockSpec((_T, _C), lambda i: (i, 0)),
            pl.BlockSpec((_C, 2 * _H), lambda i: (0, 0)),
            pl.BlockSpec((1, 2 * _H), lambda i: (0, 0)),
            pl.BlockSpec((_C, _H * _K), lambda i: (0, 0)),
            pl.BlockSpec((1, _H * _K), lambda i: (0, 0)),
        ],
        out_specs=[
            pl.BlockSpec((_T, _S * _H * 16), lambda i: (i, 0)),
            pl.BlockSpec((_T, _S), lambda i: (i, 0)),
        ],
        out_shape=[
            jax.ShapeDtypeStruct((nrows, _S * _H * 16), jnp.float32),
            jax.ShapeDtypeStruct((nrows, _S), jnp.int32),
        ],
    )(x2d, wwt, bw2, wkt, bk2)


_CH = 16
_WROW = _S * _H * 16


def _sc_body(rpw, x_hbm, idx_hbm, wexp_hbm, out_hbm, idx_ch, wexp0, wexp1,
             vals0, vals1, out_ch, semv0, semv1, semw0, semw1):
    cid = lax.axis_index("c")
    sid = lax.axis_index("s")
    wid = sid * 2 + cid
    base = wid * rpw

    z = jnp.zeros((16,), jnp.float32)
    vbufs = (vals0, vals1)
    wbufs = (wexp0, wexp1)
    vsems = (semv0, semv1)
    wsems = (semw0, semw1)

    def chunk(ci, carry):
        c0 = base + ci * _CH
        pltpu.sync_copy(idx_hbm.at[pl.ds(c0, _CH)], idx_ch)
        for q in range(2):
            pltpu.async_copy(x_hbm.at[idx_ch.at[q]], vbufs[q], vsems[q])
            pltpu.async_copy(wexp_hbm.at[c0 + q], wbufs[q], wsems[q])

        def pair(p, c2):
            for q in range(2):
                j = 2 * p + q
                vcur = vbufs[q]
                wcur = wbufs[q]
                pltpu.make_async_copy(
                    x_hbm.at[pl.ds(0, _S)], vcur, vsems[q]).wait()
                pltpu.make_async_copy(wexp_hbm.at[0], wcur, wsems[q]).wait()
                orow = out_ch.at[j]

                def hbody(h, c3, vcur=vcur, wcur=wcur, orow=orow):
                    wb = h * (_S * 16)
                    cb = h * 64
                    a0 = z
                    a1 = z
                    a2 = z
                    a3 = z
                    for s in range(_S):
                        w = wcur[pl.ds(wb + s * 16, 16)]
                        vrow = vcur.at[s]
                        e0, o0 = plsc.unpack(
                            vrow[pl.ds(cb, 32)],
                            format=plsc.PackFormat.INTERLEAVED,
                            preferred_element_type=jnp.float32)
                        e1, o1 = plsc.unpack(
                            vrow[pl.ds(cb + 32, 32)],
                            format=plsc.PackFormat.INTERLEAVED,
                            preferred_element_type=jnp.float32)
                        a0 = a0 + w * e0
                        a1 = a1 + w * o0
                        a2 = a2 + w * e1
                        a3 = a3 + w * o1
                    orow[pl.ds(cb, 16)] = a0
                    orow[pl.ds(cb + 16, 16)] = a1
                    orow[pl.ds(cb + 32, 16)] = a2
                    orow[pl.ds(cb + 48, 16)] = a3
                    return c3

                lax.fori_loop(0, _H, hbody, 0)

                @pl.when(j + 2 < _CH)
                def _():
                    pltpu.async_copy(x_hbm.at[idx_ch.at[j + 2]], vcur, vsems[q])
                    pltpu.async_copy(wexp_hbm.at[c0 + j + 2], wcur, wsems[q])
            return c2

        lax.fori_loop(0, _CH // 2, pair, 0)
        pltpu.sync_copy(out_ch, out_hbm.at[pl.ds(c0, _CH)])
        return carry

    lax.fori_loop(0, rpw // _CH, chunk, 0)


def _sc_call(x2d, idx2d, wexp2d):
    nrows = idx2d.shape[0]
    mesh = plsc.VectorSubcoreMesh(core_axis_name="c", subcore_axis_name="s")
    fn = functools.partial(
        pl.kernel,
        out_type=jax.ShapeDtypeStruct((nrows, _C), jnp.float32),
        mesh=mesh,
        scratch_types=[
            pltpu.VMEM((_CH, _S), jnp.int32),
            pltpu.VMEM((_WROW,), jnp.float32),
            pltpu.VMEM((_WROW,), jnp.float32),
            pltpu.VMEM((_S, _C), jnp.bfloat16),
            pltpu.VMEM((_S, _C), jnp.bfloat16),
            pltpu.VMEM((_CH, _C), jnp.float32),
            pltpu.SemaphoreType.DMA,
            pltpu.SemaphoreType.DMA,
            pltpu.SemaphoreType.DMA,
            pltpu.SemaphoreType.DMA,
        ],
        compiler_params=pltpu.CompilerParams(
            use_tc_tiling_on_sc=False, needs_layout_passes=False
        ),
    )(functools.partial(_sc_body, nrows // _NW))
    return fn(x2d, idx2d, wexp2d)


def _out_body(h_ref, wot_ref, o_ref):
    acc = jnp.dot(h_ref[...], wot_ref[...], preferred_element_type=jnp.float32)
    o_ref[...] = acc * jax.nn.sigmoid(acc)


def _out_call(hidden2d, wot):
    nrows = hidden2d.shape[0]
    return pl.pallas_call(
        _out_body,
        grid=(nrows // _T,),
        in_specs=[
            pl.BlockSpec((_T, _C), lambda i: (i, 0)),
            pl.BlockSpec((_C, _C), lambda i: (0, 0)),
        ],
        out_specs=pl.BlockSpec((_T, _C), lambda i: (i, 0)),
        out_shape=jax.ShapeDtypeStruct((nrows, _C), jnp.float32),
    )(hidden2d, wot)


_PERM = np.concatenate([
    np.concatenate([g * 32 + np.arange(0, 32, 2), g * 32 + np.arange(1, 32, 2)])
    for g in range(_C // 32)
])


@jax.jit
def kernel(x, Ww, bw, Wk, bk, Wo):
    x2d = x.reshape(_BL, _C)
    xbf = x2d.astype(jnp.bfloat16)
    wwt, bw2 = Ww.T, bw.reshape(1, -1)
    wkt, bk2 = Wk.T, bk.reshape(1, -1)
    wotp = Wo.T[_PERM]
    half = _BL // 2
    nb = half // _T
    wexp1, idx1 = _prep_call(x2d[:half], wwt, bw2, wkt, bk2, 0, nb)
    hid1 = _sc_call(xbf, idx1, wexp1)
    wexp2, idx2 = _prep_call(x2d[half:], wwt, bw2, wkt, bk2, nb, nb)
    hid2 = _sc_call(xbf, idx2, wexp2)
    out1 = _out_call(hid1, wotp)
    out2 = _out_call(hid2, wotp)
    return jnp.concatenate([out1, out2], axis=0).reshape(_B, _L, _C)
